# speedup vs baseline: 3.6947x; 3.6947x over previous
import jax
import jax.numpy as jnp
from jax import lax
from jax.experimental import pallas as pl
from jax.experimental.pallas import tpu as pltpu
from jax.experimental.pallas import tpu_sc as plsc

B, P, F, D = 32, 512, 4096, 256
NC, NS, L = 2, 16, 16
NW = NC * NS
C = 128
NCH = F // C


def _sc_body(x_hbm, vi_hbm, out_hbm, mask_hbm, idx_v, mask_v, buf0, buf1,
             gsem0, gsem1):
    b = lax.axis_index("s") * NC + lax.axis_index("c")
    boff = b * P

    pltpu.sync_copy(vi_hbm.at[b], idx_v)

    def pre(c, _):
        for j in range(C // L):
            sl = pl.ds(j * L, L)
            v = idx_v[c, sl]
            mask_v[c, sl] = jnp.where(v != P - 1, 1, 0)
            idx_v[c, sl] = v + boff
        return 0

    lax.fori_loop(0, NCH, pre, 0, unroll=False)

    def gather(c, buf, sem):
        return pltpu.make_async_copy(x_hbm.at[idx_v.at[c]], buf, sem)

    def write(c, buf):
        base = (b * NCH + c) * C
        pltpu.sync_copy(buf, out_hbm.at[pl.ds(base, C)])

    gather(0, buf0, gsem0).start()

    def step(i, _):
        g = 2 * i
        gather(g + 1, buf1, gsem1).start()
        gather(g, buf0, gsem0).wait()
        write(g, buf0)

        @pl.when(g + 2 < NCH)
        def _():
            gather(g + 2, buf0, gsem0).start()

        gather(g + 1, buf1, gsem1).wait()
        write(g + 1, buf1)
        return 0

    lax.fori_loop(0, NCH // 2, step, 0, unroll=False)

    pltpu.sync_copy(mask_v, mask_hbm.at[b])


def kernel(x, durations, val_ind):
    del durations
    x2 = x.reshape(B * P, D)
    vi = val_ind.astype(jnp.int32).reshape(B, NCH, C)

    mesh = plsc.VectorSubcoreMesh(core_axis_name="c", subcore_axis_name="s",
                                  num_cores=NC, num_subcores=NS)
    out_flat, mask_i32 = pl.kernel(
        _sc_body,
        out_type=[
            jax.ShapeDtypeStruct((B * F, D), jnp.float32),
            jax.ShapeDtypeStruct((B, NCH, C), jnp.int32),
        ],
        mesh=mesh,
        scratch_types=[
            pltpu.VMEM((NCH, C), jnp.int32),
            pltpu.VMEM((NCH, C), jnp.int32),
            pltpu.VMEM((C, D), jnp.float32),
            pltpu.VMEM((C, D), jnp.float32),
            pltpu.SemaphoreType.DMA,
            pltpu.SemaphoreType.DMA,
        ],
    )(x2, vi)

    return out_flat.reshape(B, F, D), mask_i32.reshape(B, F) != 0
